# baseline (speedup 1.0000x reference)
import jax
import jax.numpy as jnp
from jax.experimental import pallas as pl
from jax.experimental.pallas import tpu as pltpu

BSZ, L, D = 4, 512, 128
HEADS, OUT_CH = 2, 64


def _gat_kernel(x_ref, graph_ref, w_ref, m_ref, bias_ref, out_ref, mask_ref):
    @pl.when(pl.program_id(0) == 0)
    def _build_mask():
        ii = jax.lax.broadcasted_iota(jnp.int32, (L, L), 0)
        jj = jax.lax.broadcasted_iota(jnp.int32, (L, L), 1)
        mask_ref[...] = ((graph_ref[...] != 0.0) | (ii == jj)).astype(
            jnp.float32)

    maskf = mask_ref[...]
    x = x_ref[0]
    h = jax.lax.dot_general(x, w_ref[...], (((1,), (1,)), ((), ())),
                            preferred_element_type=jnp.float32)
    sd = jax.lax.dot_general(h, m_ref[...], (((1,), (0,)), ((), ())),
                             preferred_element_type=jnp.float32)

    ones_col = jnp.ones((L, 1), dtype=jnp.float32)
    bias = bias_ref[...]
    outs = []
    for hd in range(HEADS):
        hh1 = jnp.concatenate(
            [h[:, hd * OUT_CH:(hd + 1) * OUT_CH], ones_col], axis=1)
        s = sd[:, 2 * hd:2 * hd + 1]
        d = sd[:, 2 * hd + 1:2 * hd + 2]
        S = jnp.max(s)
        A = jnp.exp(s - S)
        C = jnp.exp(0.2 * (s - S))
        Dg = jnp.exp(-0.8 * (S + d))
        neg_dT = (-d).T
        g = jnp.broadcast_to(s, (L, L)) >= neg_dT
        E1 = jnp.where(g, maskf, 0.0)
        E2 = maskf - E1
        num1 = jax.lax.dot_general(E1, hh1 * A, (((0,), (0,)), ((), ())),
                                   preferred_element_type=jnp.float32)
        num2 = jax.lax.dot_general(E2, hh1 * C, (((0,), (0,)), ((), ())),
                                   preferred_element_type=jnp.float32)
        tot = num1 + Dg * num2
        outs.append(tot[:, :OUT_CH] / (tot[:, OUT_CH:] + 1e-16))
    out = jnp.concatenate(outs, axis=1) + bias
    out_ref[0] = jnp.maximum(out, 0.0) + x


@jax.jit
def _gat(x, graph, W, att_src, att_dst, bias):
    bias2 = bias.reshape(1, HEADS * OUT_CH)
    z = jnp.zeros((OUT_CH, 1), jnp.float32)
    m = jnp.concatenate([
        jnp.concatenate([att_src[0, :, None], att_dst[0, :, None], z, z], 1),
        jnp.concatenate([z, z, att_src[1, :, None], att_dst[1, :, None]], 1),
    ], axis=0)
    return pl.pallas_call(
        _gat_kernel,
        grid=(BSZ,),
        in_specs=[
            pl.BlockSpec((1, L, D), lambda b: (b, 0, 0)),
            pl.BlockSpec((L, L), lambda b: (0, 0)),
            pl.BlockSpec((HEADS * OUT_CH, D), lambda b: (0, 0)),
            pl.BlockSpec((HEADS * OUT_CH, 4), lambda b: (0, 0)),
            pl.BlockSpec((1, HEADS * OUT_CH), lambda b: (0, 0)),
        ],
        out_specs=pl.BlockSpec((1, L, D), lambda b: (b, 0, 0)),
        out_shape=jax.ShapeDtypeStruct((BSZ, L, HEADS * OUT_CH), jnp.float32),
        scratch_shapes=[pltpu.VMEM((L, L), jnp.float32)],
    )(x, graph, W, m, bias2)


def kernel(x, graph, W, att_src, att_dst, bias):
    return _gat(x, graph, W, att_src, att_dst, bias)


# speedup vs baseline: 1.0635x; 1.0635x over previous
import jax
import jax.numpy as jnp
from jax.experimental import pallas as pl
from jax.experimental.pallas import tpu as pltpu

BSZ, L, D = 4, 512, 128
HEADS, OUT_CH = 2, 64


def _gat_kernel(x_ref, graph_ref, w_ref, mcol_ref, mrow_ref, bias_ref,
                out_ref, mask_ref):
    @pl.when(pl.program_id(0) == 0)
    def _build_mask():
        ii = jax.lax.broadcasted_iota(jnp.int32, (L, L), 0)
        jj = jax.lax.broadcasted_iota(jnp.int32, (L, L), 1)
        mask_ref[...] = ((graph_ref[...] != 0.0) | (ii == jj)).astype(
            jnp.float32)

    maskf = mask_ref[...]
    x = x_ref[0]
    h = jax.lax.dot_general(x, w_ref[...], (((1,), (1,)), ((), ())),
                            preferred_element_type=jnp.float32)
    sd_col = jax.lax.dot_general(h, mcol_ref[...], (((1,), (0,)), ((), ())),
                                 preferred_element_type=jnp.float32)
    d_row = jax.lax.dot_general(mrow_ref[...], h, (((1,), (1,)), ((), ())),
                                preferred_element_type=jnp.float32)
    neg_d_row = -d_row
    ex = jnp.exp(sd_col)
    ones_row = jnp.ones((1, L), dtype=jnp.float32)

    ones_col = jnp.ones((L, 1), dtype=jnp.float32)
    bias = bias_ref[...]
    outs = []
    for hd in range(HEADS):
        hh1 = jnp.concatenate(
            [h[:, hd * OUT_CH:(hd + 1) * OUT_CH], ones_col], axis=1)
        k = 4 * hd
        s_bc = jax.lax.dot_general(
            sd_col[:, k:k + 1], ones_row, (((1,), (0,)), ((), ())),
            preferred_element_type=jnp.float32)
        g = s_bc >= neg_d_row[hd:hd + 1, :]
        E1 = jnp.where(g, maskf, 0.0)
        E2 = maskf - E1
        num1 = jax.lax.dot_general(
            E1, hh1 * ex[:, k:k + 1], (((0,), (0,)), ((), ())),
            preferred_element_type=jnp.float32)
        num2 = jax.lax.dot_general(
            E2, hh1 * ex[:, k + 1:k + 2], (((0,), (0,)), ((), ())),
            preferred_element_type=jnp.float32)
        tot = num1 * ex[:, k + 2:k + 3] + num2 * ex[:, k + 3:k + 4]
        outs.append(tot[:, :OUT_CH] / (tot[:, OUT_CH:] + 1e-16))
    out = jnp.concatenate(outs, axis=1) + bias
    out_ref[0] = jnp.maximum(out, 0.0) + x


@jax.jit
def _gat(x, graph, W, att_src, att_dst, bias):
    bias2 = bias.reshape(1, HEADS * OUT_CH)
    z = jnp.zeros((OUT_CH, 4), jnp.float32)
    blocks = []
    for hd in range(HEADS):
        cols = jnp.stack([att_src[hd], 0.2 * att_src[hd],
                          att_dst[hd], 0.2 * att_dst[hd]], axis=1)
        row = [z] * HEADS
        row[hd] = cols
        blocks.append(jnp.concatenate(row, axis=1))
    mcol = jnp.concatenate(blocks, axis=0)
    mrow = jnp.zeros((8, HEADS * OUT_CH), jnp.float32)
    for hd in range(HEADS):
        mrow = mrow.at[hd, hd * OUT_CH:(hd + 1) * OUT_CH].set(att_dst[hd])
    return pl.pallas_call(
        _gat_kernel,
        grid=(BSZ,),
        in_specs=[
            pl.BlockSpec((1, L, D), lambda b: (b, 0, 0)),
            pl.BlockSpec((L, L), lambda b: (0, 0)),
            pl.BlockSpec((HEADS * OUT_CH, D), lambda b: (0, 0)),
            pl.BlockSpec((HEADS * OUT_CH, 4 * HEADS), lambda b: (0, 0)),
            pl.BlockSpec((8, HEADS * OUT_CH), lambda b: (0, 0)),
            pl.BlockSpec((1, HEADS * OUT_CH), lambda b: (0, 0)),
        ],
        out_specs=pl.BlockSpec((1, L, D), lambda b: (b, 0, 0)),
        out_shape=jax.ShapeDtypeStruct((BSZ, L, HEADS * OUT_CH), jnp.float32),
        scratch_shapes=[pltpu.VMEM((L, L), jnp.float32)],
    )(x, graph, W, mcol, mrow, bias2)


def kernel(x, graph, W, att_src, att_dst, bias):
    return _gat(x, graph, W, att_src, att_dst, bias)
